# Initial kernel scaffold; baseline (speedup 1.0000x reference)
import jax
import jax.numpy as jnp
from jax.experimental import pallas as pl


def kernel(input, W0, b0, W1, b1, W2, b2, P0, q0, P1, q1):
    raise NotImplementedError("write your pallas kernel here")


# speedup vs baseline: 1.9655x; 1.9655x over previous
import jax
import jax.numpy as jnp
from jax.experimental import pallas as pl
from jax.experimental.pallas import tpu as pltpu

_TAU = 0.007
_BM = 512


def _maxprob(logits):
    m = jnp.max(logits, axis=-1, keepdims=True)
    un = jnp.exp(logits - m)
    s = jnp.sum(un, axis=-1, keepdims=True)
    return jnp.max(un / s, axis=-1)


def _dot(a, b):
    return jnp.dot(a, b, preferred_element_type=jnp.float32)


def _dense_kernel(x_ref, W0_ref, b0_ref, W1_ref, b1_ref, W2_ref, b2_ref,
                  P0_ref, q0_ref, P1_ref, q1_ref, out_ref):
    x = x_ref[...]
    h0 = jnp.maximum(_dot(x, W0_ref[...]) + b0_ref[...], 0.0)
    p0 = _dot(h0, P0_ref[...]) + q0_ref[...]
    m0 = _maxprob(p0) > _TAU
    h1 = jnp.maximum(_dot(h0, W1_ref[...]) + b1_ref[...], 0.0)
    p1 = _dot(h1, P1_ref[...]) + q1_ref[...]
    m1 = _maxprob(p1) > _TAU
    f = _dot(h1, W2_ref[...]) + b2_ref[...]
    out_ref[...] = jnp.where(m0[:, None], p0, jnp.where(m1[:, None], p1, f))


def kernel(input, W0, b0, W1, b1, W2, b2, P0, q0, P1, q1):
    B, D = input.shape
    C = P0.shape[1]
    row_spec = lambda: pl.BlockSpec((_BM, D), lambda i: (i, 0))
    full = lambda d0, d1: pl.BlockSpec((d0, d1), lambda i: (0, 0))
    vec = lambda d: pl.BlockSpec((1, d), lambda i: (0, 0))
    out = pl.pallas_call(
        _dense_kernel,
        grid=(B // _BM,),
        in_specs=[
            row_spec(),
            full(D, D), vec(D),
            full(D, D), vec(D),
            full(D, C), vec(C),
            full(D, C), vec(C),
            full(D, C), vec(C),
        ],
        out_specs=pl.BlockSpec((_BM, C), lambda i: (i, 0)),
        out_shape=jax.ShapeDtypeStruct((B, C), input.dtype),
    )(input, W0, b0.reshape(1, D), W1, b1.reshape(1, D), W2, b2.reshape(1, C),
      P0, q0.reshape(1, C), P1, q1.reshape(1, C))
    return out
